# baseline (speedup 1.0000x reference)
import functools

import jax
import jax.numpy as jnp
from jax import lax
from jax.experimental import pallas as pl
from jax.experimental.pallas import tpu as pltpu
from jax.experimental.pallas import tpu_sc as plsc

N = 10000
D = 128
E = 320000

NC = 2
NS = 16
NW = NC * NS
CH = 128
K = 80
KT = K + 4
EPW = K * CH
EPAD = EPW * NW
NPAD = 10240
ROWS_PT = NPAD // NS


def _sc_agg_body(x_hbm, srcw_hbm, dstw_hbm, zrow_hbm, zdeg_hbm,
                 agg_out, deg_out,
                 idxd_v, is0_v, is1_v, is2_v, is3_v, rows0_v, rows1_v,
                 ones_v, acc_sp, deg_sp,
                 gsem0, gsem1, ssem0, ssem1, ssem2, ssem3, osem):
    c = lax.axis_index("c")
    s = lax.axis_index("s")
    wid = c * NS + s
    base = wid * KT
    is_v = (is0_v, is1_v, is2_v, is3_v)
    ssem = (ssem0, ssem1, ssem2, ssem3)
    rows = (rows0_v, rows1_v)
    gsem = (gsem0, gsem1)
    pltpu.sync_copy(zrow_hbm, acc_sp.at[pl.ds(s * ROWS_PT, ROWS_PT)])
    pltpu.sync_copy(zdeg_hbm, deg_sp.at[pl.ds(s * ROWS_PT, ROWS_PT)])
    pltpu.sync_copy(dstw_hbm.at[wid], idxd_v)
    for i in range(CH // 16):
        ones_v[pl.ds(i * 16, 16)] = jnp.ones((16,), jnp.float32)
    plsc.subcore_barrier()

    for q in range(4):
        pltpu.async_copy(srcw_hbm.at[base + q], is_v[q], ssem[q])
    for b in range(2):
        pltpu.make_async_copy(srcw_hbm.at[base + b], is_v[b], ssem[b]).wait()
        pltpu.async_copy(x_hbm.at[is_v[b]], rows[b], gsem[b])

    def step(g, carry):
        j0 = 4 * g
        for t in range(4):
            j = j0 + t
            b = t % 2
            q = t
            qn = (t + 2) % 4
            pltpu.make_async_copy(x_hbm.at[is_v[q]], rows[b], gsem[b]).wait()
            pltpu.sync_copy(rows[b], acc_sp.at[idxd_v.at[j]], add=True)
            pltpu.sync_copy(ones_v, deg_sp.at[idxd_v.at[j]], add=True)
            pltpu.async_copy(srcw_hbm.at[base + j + 4], is_v[q], ssem[q])
            pltpu.make_async_copy(srcw_hbm.at[base + j + 2], is_v[qn],
                                  ssem[qn]).wait()
            pltpu.async_copy(x_hbm.at[is_v[qn]], rows[b], gsem[b])
        return carry

    lax.fori_loop(0, K // 4, step, 0)
    for b in range(2):
        pltpu.make_async_copy(x_hbm.at[is_v[b]], rows[b], gsem[b]).wait()
    for q in (2, 3):
        pltpu.make_async_copy(srcw_hbm.at[base + q], is_v[q], ssem[q]).wait()
    plsc.subcore_barrier()
    pltpu.sync_copy(acc_sp.at[pl.ds(s * ROWS_PT, ROWS_PT)],
                    agg_out.at[c, pl.ds(s * ROWS_PT, ROWS_PT)])
    pltpu.sync_copy(deg_sp.at[pl.ds(s * ROWS_PT, ROWS_PT)],
                    deg_out.at[c, pl.ds(s * ROWS_PT, ROWS_PT)])


_sc_agg = pl.kernel(
    _sc_agg_body,
    mesh=plsc.VectorSubcoreMesh(core_axis_name="c", subcore_axis_name="s"),
    out_type=[
        jax.ShapeDtypeStruct((NC, NPAD, D), jnp.float32),
        jax.ShapeDtypeStruct((NC, NPAD), jnp.float32),
    ],
    scratch_types=[
        pltpu.VMEM((K, CH), jnp.int32),
        pltpu.VMEM((CH,), jnp.int32),
        pltpu.VMEM((CH,), jnp.int32),
        pltpu.VMEM((CH,), jnp.int32),
        pltpu.VMEM((CH,), jnp.int32),
        pltpu.VMEM((CH, D), jnp.float32),
        pltpu.VMEM((CH, D), jnp.float32),
        pltpu.VMEM((CH,), jnp.float32),
        pltpu.VMEM_SHARED((NPAD, D), jnp.float32),
        pltpu.VMEM_SHARED((NPAD,), jnp.float32),
        pltpu.SemaphoreType.DMA,
        pltpu.SemaphoreType.DMA,
        pltpu.SemaphoreType.DMA,
        pltpu.SemaphoreType.DMA,
        pltpu.SemaphoreType.DMA,
        pltpu.SemaphoreType.DMA,
        pltpu.SemaphoreType.DMA,
    ],
)


def _layer_body(relu, h_ref, p0_ref, p1_ref, d0_ref, d1_ref,
                ws_ref, wn_ref, b_ref, o_ref):
    deg = jnp.maximum(d0_ref[...] + d1_ref[...], 1.0)
    neigh = (p0_ref[0] + p1_ref[0]) / deg
    acc = jnp.dot(h_ref[...], ws_ref[...], preferred_element_type=jnp.float32)
    acc += jnp.dot(neigh, wn_ref[...], preferred_element_type=jnp.float32)
    acc += b_ref[...]
    o_ref[...] = jnp.maximum(acc, 0.0) if relu else acc


def _tc_layer(h, aggp, d0, d1, Ws, Wn, b, relu):
    R = 400
    grid = (N // R,)
    row = pl.BlockSpec((R, D), lambda i: (i, 0))
    p0 = pl.BlockSpec((1, R, D), lambda i: (0, i, 0))
    p1 = pl.BlockSpec((1, R, D), lambda i: (1, i, 0))
    col = pl.BlockSpec((R, 1), lambda i: (i, 0))
    full = pl.BlockSpec((D, D), lambda i: (0, 0))
    bspec = pl.BlockSpec((1, D), lambda i: (0, 0))
    return pl.pallas_call(
        functools.partial(_layer_body, relu),
        grid=grid,
        in_specs=[row, p0, p1, col, col, full, full, bspec],
        out_specs=row,
        out_shape=jax.ShapeDtypeStruct((N, D), jnp.float32),
    )(h, aggp, aggp, d0, d1, Ws, Wn, b.reshape(1, D))


def kernel(h, edge_index, W_self1, W_neigh1, b1, W_self2, W_neigh2, b2):
    src = edge_index[0].astype(jnp.int32)
    dst = edge_index[1].astype(jnp.int32)
    pad = EPAD - E
    trash = N + (jnp.arange(pad, dtype=jnp.int32) % (NPAD - N))
    src_p = jnp.concatenate([src, jnp.zeros((pad,), jnp.int32)])
    dst_p = jnp.concatenate([dst, trash])
    srcw = jnp.concatenate(
        [src_p.reshape(NW, K, CH), jnp.zeros((NW, 4, CH), jnp.int32)],
        axis=1).reshape(NW * KT, CH)
    dstw = dst_p.reshape(NW, K, CH)
    zrow = jnp.zeros((ROWS_PT, D), jnp.float32)
    zdeg = jnp.zeros((ROWS_PT,), jnp.float32)

    aggp, degp = _sc_agg(h, srcw, dstw, zrow, zdeg)
    d0 = degp[0, :N, None]
    d1 = degp[1, :N, None]
    x = _tc_layer(h, aggp, d0, d1, W_self1, W_neigh1, b1, True)
    aggp2, _ = _sc_agg(x, srcw, dstw, zrow, zdeg)
    out = _tc_layer(x, aggp2, d0, d1, W_self2, W_neigh2, b2, False)
    return out


# speedup vs baseline: 2.1312x; 2.1312x over previous
import functools

import jax
import jax.numpy as jnp
from jax import lax
from jax.experimental import pallas as pl
from jax.experimental.pallas import tpu as pltpu
from jax.experimental.pallas import tpu_sc as plsc

N = 10000
D = 128
E = 320000

NC = 2
NS = 16
NW = NC * NS
CH = 128
K = 79
EPW = K * CH
EPAD = EPW * NW
NPAD = 10240
ROWS_PT = NPAD // NS


def _sc_agg_body(x_hbm, srcw_hbm, dstw_hbm, zrow_hbm, zdeg_hbm,
                 agg_out, deg_out,
                 idxs_v, idxd_v, rows_v, ones_v, acc_sp, deg_sp, sem):
    c = lax.axis_index("c")
    s = lax.axis_index("s")
    wid = c * NS + s
    pltpu.sync_copy(zrow_hbm, acc_sp.at[pl.ds(s * ROWS_PT, ROWS_PT)])
    pltpu.sync_copy(zdeg_hbm, deg_sp.at[pl.ds(s * ROWS_PT, ROWS_PT)])
    pltpu.sync_copy(srcw_hbm.at[wid], idxs_v)
    pltpu.sync_copy(dstw_hbm.at[wid], idxd_v)
    for i in range(CH // 16):
        ones_v[pl.ds(i * 16, 16)] = jnp.ones((16,), jnp.float32)
    plsc.subcore_barrier()

    def chunk(j, carry):
        pltpu.async_copy(x_hbm.at[idxs_v.at[j]], rows_v, sem).wait()
        pltpu.sync_copy(rows_v, acc_sp.at[idxd_v.at[j]], add=True)
        pltpu.sync_copy(ones_v, deg_sp.at[idxd_v.at[j]], add=True)
        return carry

    lax.fori_loop(0, K, chunk, 0)
    plsc.subcore_barrier()
    pltpu.sync_copy(acc_sp.at[pl.ds(s * ROWS_PT, ROWS_PT)],
                    agg_out.at[c, pl.ds(s * ROWS_PT, ROWS_PT)])
    pltpu.sync_copy(deg_sp.at[pl.ds(s * ROWS_PT, ROWS_PT)],
                    deg_out.at[c, pl.ds(s * ROWS_PT, ROWS_PT)])


_sc_agg = pl.kernel(
    _sc_agg_body,
    mesh=plsc.VectorSubcoreMesh(core_axis_name="c", subcore_axis_name="s"),
    out_type=[
        jax.ShapeDtypeStruct((NC, NPAD, D), jnp.float32),
        jax.ShapeDtypeStruct((NC, NPAD), jnp.float32),
    ],
    scratch_types=[
        pltpu.VMEM((K, CH), jnp.int32),
        pltpu.VMEM((K, CH), jnp.int32),
        pltpu.VMEM((CH, D), jnp.float32),
        pltpu.VMEM((CH,), jnp.float32),
        pltpu.VMEM_SHARED((NPAD, D), jnp.float32),
        pltpu.VMEM_SHARED((NPAD,), jnp.float32),
        pltpu.SemaphoreType.DMA,
    ],
)


def _layer_body(relu, h_ref, p0_ref, p1_ref, d0_ref, d1_ref,
                ws_ref, wn_ref, b_ref, o_ref):
    deg = jnp.maximum(d0_ref[...] + d1_ref[...], 1.0)
    neigh = (p0_ref[0] + p1_ref[0]) / deg
    acc = jnp.dot(h_ref[...], ws_ref[...], preferred_element_type=jnp.float32)
    acc += jnp.dot(neigh, wn_ref[...], preferred_element_type=jnp.float32)
    acc += b_ref[...]
    o_ref[...] = jnp.maximum(acc, 0.0) if relu else acc


def _tc_layer(h, aggp, d0, d1, Ws, Wn, b, relu):
    R = 400
    grid = (N // R,)
    row = pl.BlockSpec((R, D), lambda i: (i, 0))
    p0 = pl.BlockSpec((1, R, D), lambda i: (0, i, 0))
    p1 = pl.BlockSpec((1, R, D), lambda i: (1, i, 0))
    col = pl.BlockSpec((R, 1), lambda i: (i, 0))
    full = pl.BlockSpec((D, D), lambda i: (0, 0))
    bspec = pl.BlockSpec((1, D), lambda i: (0, 0))
    return pl.pallas_call(
        functools.partial(_layer_body, relu),
        grid=grid,
        in_specs=[row, p0, p1, col, col, full, full, bspec],
        out_specs=row,
        out_shape=jax.ShapeDtypeStruct((N, D), jnp.float32),
    )(h, aggp, aggp, d0, d1, Ws, Wn, b.reshape(1, D))


def kernel(h, edge_index, W_self1, W_neigh1, b1, W_self2, W_neigh2, b2):
    src = edge_index[0].astype(jnp.int32)
    dst = edge_index[1].astype(jnp.int32)
    pad = EPAD - E
    trash = N + (jnp.arange(pad, dtype=jnp.int32) % (NPAD - N))
    src_p = jnp.concatenate([src, jnp.zeros((pad,), jnp.int32)])
    dst_p = jnp.concatenate([dst, trash])
    srcw = src_p.reshape(NW, K, CH)
    dstw = dst_p.reshape(NW, K, CH)
    zrow = jnp.zeros((ROWS_PT, D), jnp.float32)
    zdeg = jnp.zeros((ROWS_PT,), jnp.float32)

    aggp, degp = _sc_agg(h, srcw, dstw, zrow, zdeg)
    d0 = degp[0, :N, None]
    d1 = degp[1, :N, None]
    x = _tc_layer(h, aggp, d0, d1, W_self1, W_neigh1, b1, True)
    aggp2, _ = _sc_agg(x, srcw, dstw, zrow, zdeg)
    out = _tc_layer(x, aggp2, d0, d1, W_self2, W_neigh2, b2, False)
    return out
